# baseline (speedup 1.0000x reference)
import functools

import jax
import jax.numpy as jnp
from jax.experimental import pallas as pl
from jax.experimental.pallas import tpu as pltpu


def _loss_kernel(x_ref, We_ref, be_ref, Wo_ref, bo_ref, out_ref):
    t = pl.program_id(1)

    @pl.when(t == 0)
    def _init():
        out_ref[...] = jnp.zeros_like(out_ref)

    x = x_ref[0]
    E = We_ref.shape[0]
    row_ids = jax.lax.broadcasted_iota(jnp.int32, out_ref.shape[1:], 0)
    acc = jnp.zeros(out_ref.shape[1:], dtype=jnp.float32)
    for e in range(E):
        h = jnp.dot(x, We_ref[e], preferred_element_type=jnp.float32)
        h = jnp.maximum(h + be_ref[e], 0.0)
        rec = jnp.dot(h, Wo_ref[e], preferred_element_type=jnp.float32)
        d = rec + bo_ref[e] - x
        acc += jnp.where(row_ids == e, jnp.sum(d * d), 0.0)
    out_ref[0] += acc


def _adapter_kernel(aidx_ref, x_ref, Wb_ref, bb_ref, Wd_ref, bd_ref,
                    Wu_ref, bu_ref, o_ref):
    del aidx_ref
    x = x_ref[0]
    base = jnp.dot(x, Wb_ref[...], preferred_element_type=jnp.float32)
    base = base + bb_ref[...]
    hid = jnp.dot(x, Wd_ref[0], preferred_element_type=jnp.float32)
    hid = jnp.maximum(hid + bd_ref[0], 0.0)
    ad = jnp.dot(hid, Wu_ref[0], preferred_element_type=jnp.float32)
    o_ref[0] = base + ad + bu_ref[0]


@functools.partial(jax.jit, static_argnames=("interpret",))
def kernel(x, W_base, b_base, Wd, bd, Wu, bu, We, be, Wo, bo, conn_idx,
           interpret=False):
    B, T, D = x.shape
    E, _, H = We.shape
    R = Wd.shape[2]
    Tt = 512
    nT = T // Tt

    sse = pl.pallas_call(
        _loss_kernel,
        grid=(B, nT),
        in_specs=[
            pl.BlockSpec((1, Tt, D), lambda b, t: (b, t, 0)),
            pl.BlockSpec((E, D, H), lambda b, t: (0, 0, 0)),
            pl.BlockSpec((E, H), lambda b, t: (0, 0)),
            pl.BlockSpec((E, H, D), lambda b, t: (0, 0, 0)),
            pl.BlockSpec((E, D), lambda b, t: (0, 0)),
        ],
        out_specs=pl.BlockSpec((1, 8, 128), lambda b, t: (b, 0, 0)),
        out_shape=jax.ShapeDtypeStruct((B, 8, 128), jnp.float32),
        interpret=interpret,
    )(x, We, be, Wo, bo)

    losses = sse[:, :E, 0]
    top1 = jnp.argmin(losses, axis=1)
    aidx = jnp.take(conn_idx, top1, axis=0).astype(jnp.int32)

    grid_spec = pltpu.PrefetchScalarGridSpec(
        num_scalar_prefetch=1,
        grid=(B, nT),
        in_specs=[
            pl.BlockSpec((1, Tt, D), lambda b, t, a: (b, t, 0)),
            pl.BlockSpec((D, D), lambda b, t, a: (0, 0)),
            pl.BlockSpec((1, D), lambda b, t, a: (0, 0)),
            pl.BlockSpec((1, D, R), lambda b, t, a: (a[b], 0, 0)),
            pl.BlockSpec((1, 1, R), lambda b, t, a: (a[b], 0, 0)),
            pl.BlockSpec((1, R, D), lambda b, t, a: (a[b], 0, 0)),
            pl.BlockSpec((1, 1, D), lambda b, t, a: (a[b], 0, 0)),
        ],
        out_specs=pl.BlockSpec((1, Tt, D), lambda b, t, a: (b, t, 0)),
    )
    out = pl.pallas_call(
        _adapter_kernel,
        grid_spec=grid_spec,
        out_shape=jax.ShapeDtypeStruct((B, T, D), jnp.float32),
        interpret=interpret,
    )(aidx, x, W_base, b_base.reshape(1, D), Wd, bd.reshape(E, 1, R),
      Wu, bu.reshape(E, 1, D))
    return out
